# baseline (speedup 1.0000x reference)
import functools
import jax
import jax.numpy as jnp
from jax import lax
from jax.experimental import pallas as pl
from jax.experimental.pallas import tpu as pltpu
from jax.experimental.pallas import tpu_sc as plsc

B, C, H, W = 256, 3, 224, 224
OW, OH = 32, 32
D = OW * OH
NC, NS = 2, 16
NW = NC * NS
PER = B // NW


def _sc_body(reg_hbm, obj_hbm, ids_hbm, tab_hbm, out_hbm,
             regb, objb, embb, idxb, sem, sem2):
    wid = lax.axis_index("s") * NC + lax.axis_index("c")
    base = wid * PER

    reg_cp = pltpu.make_async_copy(reg_hbm.at[pl.ds(base, PER)], regb, sem2)
    obj_cp = pltpu.make_async_copy(obj_hbm.at[pl.ds(base, PER)], objb, sem2)
    reg_cp.start()
    obj_cp.start()
    pltpu.sync_copy(ids_hbm.at[pl.ds(base, PER)], idxb)
    pltpu.async_copy(tab_hbm.at[idxb], embb, sem).wait()
    reg_cp.wait()
    obj_cp.wait()

    def comp(i, _):
        for j in range(D // 16):
            m = embb[i, pl.ds(16 * j, 16)]
            for c in range(C):
                r = regb[i, c, pl.ds(16 * j, 16)]
                o = objb[i, c, pl.ds(16 * j, 16)]
                regb[i, c, pl.ds(16 * j, 16)] = r + m * (o - r)
        return 0
    lax.fori_loop(0, PER, comp, 0)

    pltpu.sync_copy(regb, out_hbm.at[pl.ds(base, PER)])


def kernel(obj, bg, coord, obj_id, table):
    x = coord[0]
    y = coord[1]
    zero = jnp.zeros((), dtype=coord.dtype)
    region = lax.dynamic_slice(bg, (zero, zero, x, y), (B, C, OW, OH))
    regf = region.reshape(B, C, D)
    objf = obj.reshape(B, C, D)

    mesh = plsc.VectorSubcoreMesh(
        core_axis_name="c", subcore_axis_name="s",
        num_cores=NC, num_subcores=NS)

    sc = functools.partial(
        pl.kernel,
        mesh=mesh,
        out_type=jax.ShapeDtypeStruct((B, C, D), jnp.float32),
        scratch_types=[
            pltpu.VMEM((PER, C, D), jnp.float32),
            pltpu.VMEM((PER, C, D), jnp.float32),
            pltpu.VMEM((PER, D), jnp.float32),
            pltpu.VMEM((PER,), jnp.int32),
            pltpu.SemaphoreType.DMA,
            pltpu.SemaphoreType.DMA,
        ],
    )(_sc_body)

    newf = sc(regf, objf, obj_id, table)
    new_region = newf.reshape(B, C, OW, OH)
    return lax.dynamic_update_slice(bg, new_region, (zero, zero, x, y))
